# baseline (speedup 1.0000x reference)
import functools

import jax
import jax.numpy as jnp
from jax import lax
from jax.experimental import pallas as pl
from jax.experimental.pallas import tpu as pltpu
from jax.experimental.pallas import tpu_sc as plsc

K = 64
N = 32768
ROWS = 128
L = 16
NVEC = N // L
NG1 = NVEC // 8
NG2 = NG1 // 8
PRUNE_AT = 192
CAP = 320
NWORKERS = 32
ROWS_PER = ROWS // NWORKERS
MINI = -2147483648


def _iota():
    return lax.iota(jnp.int32, L)


def _rot(v, s):
    return v.at[(_iota() + s) & (L - 1)].get(mode="promise_in_bounds")


def _tsum(x):
    for s in (8, 4, 2, 1):
        x = x + _rot(x, s)
    return x[0]


def _tmax(x):
    for s in (8, 4, 2, 1):
        x = jnp.maximum(x, _rot(x, s))
    return x[0]


def _splat_i(s):
    return jnp.zeros((L,), jnp.int32) + s


def _splat_f(s):
    return jnp.zeros((L,), jnp.float32) + s


def _fkey(v):
    i = lax.bitcast_convert_type(v, jnp.int32)
    return i ^ (lax.shift_right_arithmetic(i, 31) & jnp.int32(0x7FFFFFFF))


def _ds16(i):
    return pl.ds(pl.multiple_of(i * L, L), L)


def _packed_mask(m):
    one = jnp.int32(1)
    return _tsum(jnp.where(m > 0, lax.shift_left(one, _iota()) | (one << 16),
                           0))


def _ffs(mb):
    low = mb & (-mb)
    f = lax.convert_element_type(low, jnp.float32)
    return (lax.shift_right_arithmetic(
        lax.bitcast_convert_type(f, jnp.int32), 23) - 127)


def _sc_body(x_hbm, oval_hbm, oidx_hbm,
             row_v, s1f, s2f, kbuf, cval, cidx, rbuf, sbuf, oval, oidx,
             g1max, g2max):
    wid = lax.axis_index("s") * 2 + lax.axis_index("c")
    iota = _iota()

    def bitsearch(keys_ref, nv_static, cnt, nbits=32):
        def bit_step(b, pu):
            trial = pu | lax.shift_left(jnp.int32(1), 31 - b)
            cand = trial ^ jnp.int32(MINI)
            acc = jnp.zeros((L,), jnp.int32)
            for v in range(nv_static):
                kv = keys_ref[_ds16(v)]
                ge = jnp.where(kv >= cand, 1, 0)
                va = jnp.where((iota + v * L) < cnt, 1, 0)
                acc = acc + ge * va
            return jnp.where(_tsum(acc) >= K, trial, pu)

        pu = lax.fori_loop(0, nbits, bit_step, jnp.int32(0))
        return pu ^ jnp.int32(MINI)

    def prune(cnt):
        for v in range(CAP // L):
            kbuf[_ds16(v)] = _fkey(cval[_ds16(v)])
        T = bitsearch(kbuf, CAP // L, cnt)

        acc = jnp.zeros((L,), jnp.int32)
        for v in range(CAP // L):
            kv = kbuf[_ds16(v)]
            gt = jnp.where(kv > T, 1, 0)
            va = jnp.where((iota + v * L) < cnt, 1, 0)
            acc = acc + gt * va
        quota = K - _tsum(acc)

        def comp_vec(v, st):
            kv = kbuf[_ds16(v)]
            fv = cval[_ds16(v)]
            iv = cidx[_ds16(v)]
            ge = jnp.where(kv >= T, 1, 0)
            va = jnp.where((iota + v * L) < cnt, 1, 0)
            msel = ge * va
            packed = _packed_mask(msel)
            nsel = lax.shift_right_arithmetic(packed, 16)

            def lane_step(t, st2):
                ncnt, eqt, thr2, mb = st2
                lane = _ffs(mb)
                mb = mb & (mb - 1)
                ks = _rot(kv, lane)[0]
                is_eq = ks == T
                take = jnp.logical_or(jnp.logical_not(is_eq), eqt < quota)

                def do_take(s3):
                    ncnt, thr2 = s3
                    val = _rot(fv, lane)[0]
                    idx = _rot(iv, lane)[0]
                    ins = ncnt & (L - 1)
                    cv = cval[_ds16(ncnt >> 4)]
                    cval[_ds16(ncnt >> 4)] = jnp.where(
                        iota == ins, _splat_f(val), cv)
                    ci = cidx[_ds16(ncnt >> 4)]
                    cidx[_ds16(ncnt >> 4)] = jnp.where(
                        iota == ins, _splat_i(idx), ci)
                    thr2 = jnp.where(is_eq, val, thr2)
                    return ncnt + 1, thr2

                ncnt, thr2 = lax.cond(take, do_take, lambda s3: s3,
                                      (ncnt, thr2))
                eqt = eqt + jnp.where(is_eq, 1, 0)
                return ncnt, eqt, thr2, mb

            ncnt, eqt, thr2 = st
            mb = packed & jnp.int32(0xFFFF)
            ncnt, eqt, thr2, _ = lax.fori_loop(
                0, nsel, lane_step, (ncnt, eqt, thr2, mb))
            return ncnt, eqt, thr2

        ncnt, _, thr_new = lax.fori_loop(
            0, CAP // L, comp_vec,
            (jnp.int32(0), jnp.int32(0), jnp.float32(0.0)))
        return ncnt, thr_new

    def append_hits(v, base, st):
        cnt, thr = st
        m = jnp.where(v >= _splat_f(thr), 1, 0)
        packed = _packed_mask(m)
        nsel = lax.shift_right_arithmetic(packed, 16)

        def lane_step(t, st2):
            mb, cnt, thr = st2
            lane = _ffs(mb)
            mb = mb & (mb - 1)
            val = _rot(v, lane)[0]
            ins = cnt & (L - 1)
            cv = cval[_ds16(cnt >> 4)]
            cval[_ds16(cnt >> 4)] = jnp.where(iota == ins, _splat_f(val), cv)
            ci = cidx[_ds16(cnt >> 4)]
            cidx[_ds16(cnt >> 4)] = jnp.where(iota == ins,
                                              _splat_i(base + lane), ci)
            return mb, cnt + 1, thr

        _, cnt, thr = lax.fori_loop(
            0, nsel, lane_step, (packed & jnp.int32(0xFFFF), cnt, thr))
        return cnt, thr

    def do_row(r, _):
        row = wid * ROWS_PER + r
        pltpu.sync_copy(x_hbm.at[row], row_v)

        def g1_step(jq, _):
            accs = []
            for u in range(4):
                j = jq * 4 + u
                acc = row_v[_ds16(j * 8)]
                for t in range(1, 8):
                    acc = jnp.maximum(acc, row_v[_ds16(j * 8 + t)])
                accs.append(acc)
            for u in range(4):
                j = jq * 4 + u
                s1f[_ds16(j)] = accs[u]
                g1max[j] = _tmax(accs[u])
            return 0

        lax.fori_loop(0, NG1 // 4, g1_step, 0)

        def g2_step(q, _):
            acc = s1f[_ds16(q * 8)]
            for t in range(1, 8):
                acc = jnp.maximum(acc, s1f[_ds16(q * 8 + t)])
            s2f[_ds16(q)] = acc
            kbuf[_ds16(q)] = _fkey(acc)
            g2max[q] = _tmax(acc)
            return 0

        lax.fori_loop(0, NG2, g2_step, 0)

        T0 = bitsearch(kbuf, NG2, jnp.int32(NG2 * L), nbits=16)
        accf = jnp.full((L,), jnp.inf, jnp.float32)
        for q in range(NG2):
            m = kbuf[_ds16(q)] >= T0
            accf = jnp.minimum(accf, jnp.where(m, s2f[_ds16(q)], jnp.inf))
        thr0 = -_tmax(-accf)

        def scan_g1(j, st):
            def descend(st2):
                s1v = s1f[_ds16(j)]
                hp = _packed_mask(jnp.where(s1v >= _splat_f(st2[1]), 1, 0))
                nl = lax.shift_right_arithmetic(hp, 16)

                def one_lane(st3):
                    cnt, thr = st3
                    lane = _ffs(hp & jnp.int32(0xFFFF))
                    perm = (iota + lane) & (L - 1)
                    bits = jnp.int32(0)
                    nh = jnp.int32(0)
                    for t in range(8):
                        val = row_v[_ds16(j * 8 + t)].at[perm].get(
                            mode="promise_in_bounds")[0]
                        hit = jnp.where(val >= thr, 1, 0)
                        bits = bits | lax.shift_left(hit, t)
                        nh = nh + hit

                    def hit_step(h, st4):
                        bits, cnt = st4
                        t = _ffs(bits)
                        bits = bits & (bits - 1)
                        val = row_v[_ds16(j * 8 + t)].at[perm].get(
                            mode="promise_in_bounds")[0]
                        ins = cnt & (L - 1)
                        cv = cval[_ds16(cnt >> 4)]
                        cval[_ds16(cnt >> 4)] = jnp.where(
                            iota == ins, _splat_f(val), cv)
                        ci = cidx[_ds16(cnt >> 4)]
                        cidx[_ds16(cnt >> 4)] = jnp.where(
                            iota == ins,
                            _splat_i(j * 128 + t * L + lane), ci)
                        return bits, cnt + 1

                    _, cnt = lax.fori_loop(0, nh, hit_step, (bits, cnt))
                    return cnt, thr

                def multi_lane(st3):
                    for t in range(8):
                        v = row_v[_ds16(j * 8 + t)]
                        st3 = append_hits(v, j * 128 + t * L, st3)
                    return st3

                st2 = lax.cond(nl == 1, one_lane, multi_lane, st2)
                return lax.cond(st2[0] >= PRUNE_AT,
                                lambda s5: prune(s5[0]),
                                lambda s5: s5, st2)

            return lax.cond(g1max[j] >= st[1], descend, lambda s: s, st)

        def scan_g2(q, st):
            return lax.cond(
                g2max[q] >= st[1],
                lambda s: lax.fori_loop(q * 8, q * 8 + 8, scan_g1, s),
                lambda s: s, st)

        cnt, thr = lax.fori_loop(0, NG2, scan_g2, (jnp.int32(0), thr0))
        cnt, thr = prune(cnt)

        for a in range(4):
            rbuf[_ds16(a)] = jnp.zeros((L,), jnp.int32)

        def rank_step(j, _):
            fv = cval[_ds16(j >> 4)]
            fj = _splat_f(_rot(fv, j & (L - 1))[0])
            for a in range(4):
                fa = cval[_ds16(a)]
                pos = iota + a * L
                gt = jnp.where(fj > fa, 1, 0)
                eq = jnp.where(fj == fa, 1, 0)
                lt = jnp.where(pos > j, 1, 0)
                rbuf[_ds16(a)] = rbuf[_ds16(a)] + gt + eq * lt
            return 0

        lax.fori_loop(0, K, rank_step, 0)

        def inv_step(j, _):
            rv = rbuf[_ds16(j >> 4)]
            rj = _rot(rv, j & (L - 1))[0]
            lane_m = jnp.where(iota == _splat_i(rj & (L - 1)), 1, 0)
            for a in range(4):
                ina = jnp.where(lax.shift_right_arithmetic(rj, 4) == a, 1, 0)
                sv = sbuf[_ds16(a)]
                sbuf[_ds16(a)] = jnp.where(lane_m * ina > 0,
                                           _splat_i(j), sv)
            return 0

        lax.fori_loop(0, K, inv_step, 0)

        for a in range(4):
            S = sbuf[_ds16(a)]
            src_lane = S & (L - 1)
            src_vec = lax.shift_right_arithmetic(S, 4)
            ov = jnp.zeros((L,), jnp.float32)
            oi_ = jnp.zeros((L,), jnp.int32)
            for b in range(4):
                g = cval[_ds16(b)].at[src_lane].get(mode="promise_in_bounds")
                gi = cidx[_ds16(b)].at[src_lane].get(mode="promise_in_bounds")
                ov = jnp.where(src_vec == b, g, ov)
                oi_ = jnp.where(src_vec == b, gi, oi_)
            oval[_ds16(a)] = ov
            oidx[_ds16(a)] = oi_

        pltpu.sync_copy(oval, oval_hbm.at[row])
        pltpu.sync_copy(oidx, oidx_hbm.at[row])
        return 0

    lax.fori_loop(0, ROWS_PER, do_row, 0)


_sc_topk = functools.partial(
    pl.kernel,
    out_type=[jax.ShapeDtypeStruct((ROWS, K), jnp.float32),
              jax.ShapeDtypeStruct((ROWS, K), jnp.int32)],
    mesh=plsc.VectorSubcoreMesh(core_axis_name="c", subcore_axis_name="s",
                                num_cores=2, num_subcores=16),
    scratch_types=[
        pltpu.VMEM((N,), jnp.float32),
        pltpu.VMEM((NG1 * L,), jnp.float32),
        pltpu.VMEM((NG2 * L,), jnp.float32),
        pltpu.VMEM((NG2 * L,), jnp.int32),
        pltpu.VMEM((CAP,), jnp.float32),
        pltpu.VMEM((CAP,), jnp.int32),
        pltpu.VMEM((K,), jnp.int32),
        pltpu.VMEM((K,), jnp.int32),
        pltpu.VMEM((K,), jnp.float32),
        pltpu.VMEM((K,), jnp.int32),
        pltpu.SMEM((NG1,), jnp.float32),
        pltpu.SMEM((NG2,), jnp.float32),
    ],
)(_sc_body)


def kernel(input):
    vals, idx = _sc_topk(input)
    return (vals, idx)


# speedup vs baseline: 1.0430x; 1.0430x over previous
import functools

import jax
import jax.numpy as jnp
from jax import lax
from jax.experimental import pallas as pl
from jax.experimental.pallas import tpu as pltpu
from jax.experimental.pallas import tpu_sc as plsc

K = 64
N = 32768
ROWS = 128
L = 16
NVEC = N // L
NG1 = NVEC // 8
NG2 = NG1 // 8
PRUNE_AT = 192
CAP = 320
NWORKERS = 32
ROWS_PER = ROWS // NWORKERS
MINI = -2147483648


def _iota():
    return lax.iota(jnp.int32, L)


def _rot(v, s):
    return v.at[(_iota() + s) & (L - 1)].get(mode="promise_in_bounds")


def _tsum(x):
    for s in (8, 4, 2, 1):
        x = x + _rot(x, s)
    return x[0]


def _tmax(x):
    for s in (8, 4, 2, 1):
        x = jnp.maximum(x, _rot(x, s))
    return x[0]


def _splat_i(s):
    return jnp.zeros((L,), jnp.int32) + s


def _splat_f(s):
    return jnp.zeros((L,), jnp.float32) + s


def _fkey(v):
    i = lax.bitcast_convert_type(v, jnp.int32)
    return i ^ (lax.shift_right_arithmetic(i, 31) & jnp.int32(0x7FFFFFFF))


def _ds16(i):
    return pl.ds(pl.multiple_of(i * L, L), L)


def _packed_mask(m):
    one = jnp.int32(1)
    return _tsum(jnp.where(m > 0, lax.shift_left(one, _iota()) | (one << 16),
                           0))


def _ffs(mb):
    low = mb & (-mb)
    f = lax.convert_element_type(low, jnp.float32)
    return (lax.shift_right_arithmetic(
        lax.bitcast_convert_type(f, jnp.int32), 23) - 127)


def _sc_body(x_hbm, oval_hbm, oidx_hbm,
             row_v, s1f, s2f, kbuf, cval, cidx, rbuf, sbuf, oval, oidx,
             gmaxv):
    wid = lax.axis_index("s") * 2 + lax.axis_index("c")
    iota = _iota()

    def bitsearch(keys_ref, nv_static, cnt, nbits=32):
        def bit_step(b, pu):
            trial = pu | lax.shift_left(jnp.int32(1), 31 - b)
            cand = trial ^ jnp.int32(MINI)
            acc = jnp.zeros((L,), jnp.int32)
            for v in range(nv_static):
                kv = keys_ref[_ds16(v)]
                ge = jnp.where(kv >= cand, 1, 0)
                va = jnp.where((iota + v * L) < cnt, 1, 0)
                acc = acc + ge * va
            return jnp.where(_tsum(acc) >= K, trial, pu)

        pu = lax.fori_loop(0, nbits, bit_step, jnp.int32(0))
        return pu ^ jnp.int32(MINI)

    def prune(cnt):
        for v in range(CAP // L):
            kbuf[_ds16(v)] = _fkey(cval[_ds16(v)])
        T = bitsearch(kbuf, CAP // L, cnt)

        acc = jnp.zeros((L,), jnp.int32)
        for v in range(CAP // L):
            kv = kbuf[_ds16(v)]
            gt = jnp.where(kv > T, 1, 0)
            va = jnp.where((iota + v * L) < cnt, 1, 0)
            acc = acc + gt * va
        quota = K - _tsum(acc)

        def comp_vec(v, st):
            kv = kbuf[_ds16(v)]
            fv = cval[_ds16(v)]
            iv = cidx[_ds16(v)]
            ge = jnp.where(kv >= T, 1, 0)
            va = jnp.where((iota + v * L) < cnt, 1, 0)
            msel = ge * va
            packed = _packed_mask(msel)
            nsel = lax.shift_right_arithmetic(packed, 16)

            def lane_step(t, st2):
                ncnt, eqt, thr2, mb = st2
                lane = _ffs(mb)
                mb = mb & (mb - 1)
                ks = _rot(kv, lane)[0]
                is_eq = ks == T
                take = jnp.logical_or(jnp.logical_not(is_eq), eqt < quota)

                def do_take(s3):
                    ncnt, thr2 = s3
                    val = _rot(fv, lane)[0]
                    idx = _rot(iv, lane)[0]
                    ins = ncnt & (L - 1)
                    cv = cval[_ds16(ncnt >> 4)]
                    cval[_ds16(ncnt >> 4)] = jnp.where(
                        iota == ins, _splat_f(val), cv)
                    ci = cidx[_ds16(ncnt >> 4)]
                    cidx[_ds16(ncnt >> 4)] = jnp.where(
                        iota == ins, _splat_i(idx), ci)
                    thr2 = jnp.where(is_eq, val, thr2)
                    return ncnt + 1, thr2

                ncnt, thr2 = lax.cond(take, do_take, lambda s3: s3,
                                      (ncnt, thr2))
                eqt = eqt + jnp.where(is_eq, 1, 0)
                return ncnt, eqt, thr2, mb

            ncnt, eqt, thr2 = st
            mb = packed & jnp.int32(0xFFFF)
            ncnt, eqt, thr2, _ = lax.fori_loop(
                0, nsel, lane_step, (ncnt, eqt, thr2, mb))
            return ncnt, eqt, thr2

        ncnt, _, thr_new = lax.fori_loop(
            0, CAP // L, comp_vec,
            (jnp.int32(0), jnp.int32(0), jnp.float32(0.0)))
        return ncnt, thr_new

    def append_hits(v, base, st):
        cnt, thr = st
        m = jnp.where(v >= _splat_f(thr), 1, 0)
        packed = _packed_mask(m)
        nsel = lax.shift_right_arithmetic(packed, 16)

        def lane_step(t, st2):
            mb, cnt, thr = st2
            lane = _ffs(mb)
            mb = mb & (mb - 1)
            val = _rot(v, lane)[0]
            ins = cnt & (L - 1)
            cv = cval[_ds16(cnt >> 4)]
            cval[_ds16(cnt >> 4)] = jnp.where(iota == ins, _splat_f(val), cv)
            ci = cidx[_ds16(cnt >> 4)]
            cidx[_ds16(cnt >> 4)] = jnp.where(iota == ins,
                                              _splat_i(base + lane), ci)
            return mb, cnt + 1, thr

        _, cnt, thr = lax.fori_loop(
            0, nsel, lane_step, (packed & jnp.int32(0xFFFF), cnt, thr))
        return cnt, thr

    def do_row(r, _):
        row = wid * ROWS_PER + r
        pltpu.sync_copy(x_hbm.at[row], row_v)

        def g1_step(jq, _):
            accs = []
            for u in range(4):
                j = jq * 4 + u
                acc = row_v[_ds16(j * 8)]
                for t in range(1, 8):
                    acc = jnp.maximum(acc, row_v[_ds16(j * 8 + t)])
                accs.append(acc)
            gv = gmaxv[_ds16(jq >> 2)]
            for u in range(4):
                j = jq * 4 + u
                s1f[_ds16(j)] = accs[u]
                lane = (jq & 3) * 4 + u
                gv = jnp.where(iota == lane, _splat_f(_tmax(accs[u])), gv)
            gmaxv[_ds16(jq >> 2)] = gv
            return 0

        lax.fori_loop(0, NG1 // 4, g1_step, 0)

        def g2_step(q, _):
            acc = s1f[_ds16(q * 8)]
            for t in range(1, 8):
                acc = jnp.maximum(acc, s1f[_ds16(q * 8 + t)])
            s2f[_ds16(q)] = acc
            kbuf[_ds16(q)] = _fkey(acc)
            return 0

        lax.fori_loop(0, NG2, g2_step, 0)

        T0 = bitsearch(kbuf, NG2, jnp.int32(NG2 * L), nbits=16)
        accf = jnp.full((L,), jnp.inf, jnp.float32)
        for q in range(NG2):
            m = kbuf[_ds16(q)] >= T0
            accf = jnp.minimum(accf, jnp.where(m, s2f[_ds16(q)], jnp.inf))
        thr0 = -_tmax(-accf)

        def descend_group(j, st2):
            s1v = s1f[_ds16(j)]
            hp = _packed_mask(jnp.where(s1v >= _splat_f(st2[1]), 1, 0))
            nl = lax.shift_right_arithmetic(hp, 16)

            def one_lane(st3):
                cnt, thr = st3
                lane = _ffs(hp & jnp.int32(0xFFFF))
                perm = (iota + lane) & (L - 1)
                bits = jnp.int32(0)
                nh = jnp.int32(0)
                for t in range(8):
                    val = row_v[_ds16(j * 8 + t)].at[perm].get(
                        mode="promise_in_bounds")[0]
                    hit = jnp.where(val >= thr, 1, 0)
                    bits = bits | lax.shift_left(hit, t)
                    nh = nh + hit

                def hit_step(h, st4):
                    bits, cnt = st4
                    t = _ffs(bits)
                    bits = bits & (bits - 1)
                    val = row_v[_ds16(j * 8 + t)].at[perm].get(
                        mode="promise_in_bounds")[0]
                    ins = cnt & (L - 1)
                    cv = cval[_ds16(cnt >> 4)]
                    cval[_ds16(cnt >> 4)] = jnp.where(
                        iota == ins, _splat_f(val), cv)
                    ci = cidx[_ds16(cnt >> 4)]
                    cidx[_ds16(cnt >> 4)] = jnp.where(
                        iota == ins,
                        _splat_i(j * 128 + t * L + lane), ci)
                    return bits, cnt + 1

                _, cnt = lax.fori_loop(0, nh, hit_step, (bits, cnt))
                return cnt, thr

            def multi_lane(st3):
                for t in range(8):
                    v = row_v[_ds16(j * 8 + t)]
                    st3 = append_hits(v, j * 128 + t * L, st3)
                return st3

            st2 = lax.cond(nl == 1, one_lane, multi_lane, st2)
            return lax.cond(st2[0] >= PRUNE_AT,
                        lambda s5: prune(s5[0]),
                        lambda s5: s5, st2)


        def scan_block(gq, st):
            gv = gmaxv[_ds16(gq)]
            m = jnp.where(gv >= _splat_f(st[1]), 1, 0)
            packed = _packed_mask(m)
            nt = lax.shift_right_arithmetic(packed, 16)

            def grp_step(h, st2):
                bits, cnt, thr = st2
                l = _ffs(bits)
                bits = bits & (bits - 1)
                cnt, thr = descend_group(gq * L + l, (cnt, thr))
                return bits, cnt, thr

            _, cnt, thr = lax.fori_loop(
                0, nt, grp_step,
                (packed & jnp.int32(0xFFFF), st[0], st[1]))
            return cnt, thr

        cnt, thr = lax.fori_loop(0, NG1 // L, scan_block,
                                 (jnp.int32(0), thr0))
        cnt, thr = prune(cnt)

        for a in range(4):
            rbuf[_ds16(a)] = jnp.zeros((L,), jnp.int32)

        def rank_step(j, _):
            fv = cval[_ds16(j >> 4)]
            fj = _splat_f(_rot(fv, j & (L - 1))[0])
            for a in range(4):
                fa = cval[_ds16(a)]
                pos = iota + a * L
                gt = jnp.where(fj > fa, 1, 0)
                eq = jnp.where(fj == fa, 1, 0)
                lt = jnp.where(pos > j, 1, 0)
                rbuf[_ds16(a)] = rbuf[_ds16(a)] + gt + eq * lt
            return 0

        lax.fori_loop(0, K, rank_step, 0)

        def inv_step(j, _):
            rv = rbuf[_ds16(j >> 4)]
            rj = _rot(rv, j & (L - 1))[0]
            lane_m = jnp.where(iota == _splat_i(rj & (L - 1)), 1, 0)
            for a in range(4):
                ina = jnp.where(lax.shift_right_arithmetic(rj, 4) == a, 1, 0)
                sv = sbuf[_ds16(a)]
                sbuf[_ds16(a)] = jnp.where(lane_m * ina > 0,
                                           _splat_i(j), sv)
            return 0

        lax.fori_loop(0, K, inv_step, 0)

        for a in range(4):
            S = sbuf[_ds16(a)]
            src_lane = S & (L - 1)
            src_vec = lax.shift_right_arithmetic(S, 4)
            ov = jnp.zeros((L,), jnp.float32)
            oi_ = jnp.zeros((L,), jnp.int32)
            for b in range(4):
                g = cval[_ds16(b)].at[src_lane].get(mode="promise_in_bounds")
                gi = cidx[_ds16(b)].at[src_lane].get(mode="promise_in_bounds")
                ov = jnp.where(src_vec == b, g, ov)
                oi_ = jnp.where(src_vec == b, gi, oi_)
            oval[_ds16(a)] = ov
            oidx[_ds16(a)] = oi_

        pltpu.sync_copy(oval, oval_hbm.at[row])
        pltpu.sync_copy(oidx, oidx_hbm.at[row])
        return 0

    lax.fori_loop(0, ROWS_PER, do_row, 0)


_sc_topk = functools.partial(
    pl.kernel,
    out_type=[jax.ShapeDtypeStruct((ROWS, K), jnp.float32),
              jax.ShapeDtypeStruct((ROWS, K), jnp.int32)],
    mesh=plsc.VectorSubcoreMesh(core_axis_name="c", subcore_axis_name="s",
                                num_cores=2, num_subcores=16),
    scratch_types=[
        pltpu.VMEM((N,), jnp.float32),
        pltpu.VMEM((NG1 * L,), jnp.float32),
        pltpu.VMEM((NG2 * L,), jnp.float32),
        pltpu.VMEM((NG2 * L,), jnp.int32),
        pltpu.VMEM((CAP,), jnp.float32),
        pltpu.VMEM((CAP,), jnp.int32),
        pltpu.VMEM((K,), jnp.int32),
        pltpu.VMEM((K,), jnp.int32),
        pltpu.VMEM((K,), jnp.float32),
        pltpu.VMEM((K,), jnp.int32),
        pltpu.VMEM((NG1,), jnp.float32),
    ],
)(_sc_body)


def kernel(input):
    vals, idx = _sc_topk(input)
    return (vals, idx)


# speedup vs baseline: 1.0767x; 1.0323x over previous
import functools

import jax
import jax.numpy as jnp
from jax import lax
from jax.experimental import pallas as pl
from jax.experimental.pallas import tpu as pltpu
from jax.experimental.pallas import tpu_sc as plsc

K = 64
N = 32768
ROWS = 128
L = 16
NVEC = N // L
NG1 = NVEC // 8
NG2 = NG1 // 8
PRUNE_AT = 192
CAP = 320
NWORKERS = 32
ROWS_PER = ROWS // NWORKERS
MINI = -2147483648


def _iota():
    return lax.iota(jnp.int32, L)


def _rot(v, s):
    return v.at[(_iota() + s) & (L - 1)].get(mode="promise_in_bounds")


def _tsum(x):
    for s in (8, 4, 2, 1):
        x = x + _rot(x, s)
    return x[0]


def _tmax(x):
    for s in (8, 4, 2, 1):
        x = jnp.maximum(x, _rot(x, s))
    return x[0]


def _splat_i(s):
    return jnp.zeros((L,), jnp.int32) + s


def _splat_f(s):
    return jnp.zeros((L,), jnp.float32) + s


def _fkey(v):
    i = lax.bitcast_convert_type(v, jnp.int32)
    return i ^ (lax.shift_right_arithmetic(i, 31) & jnp.int32(0x7FFFFFFF))


def _ds16(i):
    return pl.ds(pl.multiple_of(i * L, L), L)


def _packed_mask(m):
    one = jnp.int32(1)
    return _tsum(jnp.where(m > 0, lax.shift_left(one, _iota()) | (one << 16),
                           0))


def _ffs(mb):
    low = mb & (-mb)
    f = lax.convert_element_type(low, jnp.float32)
    return (lax.shift_right_arithmetic(
        lax.bitcast_convert_type(f, jnp.int32), 23) - 127)


def _sc_body(x_hbm, oval_hbm, oidx_hbm,
             row_v, s1f, s2f, kbuf, cval, cidx, rbuf, sbuf, oval, oidx,
             gmaxv):
    wid = lax.axis_index("s") * 2 + lax.axis_index("c")
    iota = _iota()

    def bitsearch(keys_ref, nv_static, cnt, nbits=32):
        def bit_step(b, pu):
            trial = pu | lax.shift_left(jnp.int32(1), 31 - b)
            cand = trial ^ jnp.int32(MINI)
            acc = jnp.zeros((L,), jnp.int32)
            for v in range(nv_static):
                kv = keys_ref[_ds16(v)]
                ge = jnp.where(kv >= cand, 1, 0)
                va = jnp.where((iota + v * L) < cnt, 1, 0)
                acc = acc + ge * va
            return jnp.where(_tsum(acc) >= K, trial, pu)

        pu = lax.fori_loop(0, nbits, bit_step, jnp.int32(0))
        return pu ^ jnp.int32(MINI)

    def prune(cnt, nv=CAP // L):
        for v in range(nv):
            kbuf[_ds16(v)] = _fkey(cval[_ds16(v)])
        T = bitsearch(kbuf, nv, cnt)

        acc = jnp.zeros((L,), jnp.int32)
        for v in range(nv):
            kv = kbuf[_ds16(v)]
            gt = jnp.where(kv > T, 1, 0)
            va = jnp.where((iota + v * L) < cnt, 1, 0)
            acc = acc + gt * va
        quota = K - _tsum(acc)

        def comp_vec(v, st):
            kv = kbuf[_ds16(v)]
            fv = cval[_ds16(v)]
            iv = cidx[_ds16(v)]
            ge = jnp.where(kv >= T, 1, 0)
            va = jnp.where((iota + v * L) < cnt, 1, 0)
            msel = ge * va
            packed = _packed_mask(msel)
            nsel = lax.shift_right_arithmetic(packed, 16)

            def lane_step(t, st2):
                ncnt, eqt, thr2, mb = st2
                lane = _ffs(mb)
                mb = mb & (mb - 1)
                ks = _rot(kv, lane)[0]
                is_eq = ks == T
                take = jnp.logical_or(jnp.logical_not(is_eq), eqt < quota)

                def do_take(s3):
                    ncnt, thr2 = s3
                    val = _rot(fv, lane)[0]
                    idx = _rot(iv, lane)[0]
                    ins = ncnt & (L - 1)
                    cv = cval[_ds16(ncnt >> 4)]
                    cval[_ds16(ncnt >> 4)] = jnp.where(
                        iota == ins, _splat_f(val), cv)
                    ci = cidx[_ds16(ncnt >> 4)]
                    cidx[_ds16(ncnt >> 4)] = jnp.where(
                        iota == ins, _splat_i(idx), ci)
                    thr2 = jnp.where(is_eq, val, thr2)
                    return ncnt + 1, thr2

                ncnt, thr2 = lax.cond(take, do_take, lambda s3: s3,
                                      (ncnt, thr2))
                eqt = eqt + jnp.where(is_eq, 1, 0)
                return ncnt, eqt, thr2, mb

            ncnt, eqt, thr2 = st
            mb = packed & jnp.int32(0xFFFF)
            ncnt, eqt, thr2, _ = lax.fori_loop(
                0, nsel, lane_step, (ncnt, eqt, thr2, mb))
            return ncnt, eqt, thr2

        ncnt, _, thr_new = lax.fori_loop(
            0, nv, comp_vec,
            (jnp.int32(0), jnp.int32(0), jnp.float32(0.0)))
        return ncnt, thr_new

    def append_hits(v, base, st):
        cnt, thr = st
        m = jnp.where(v >= _splat_f(thr), 1, 0)
        packed = _packed_mask(m)
        nsel = lax.shift_right_arithmetic(packed, 16)

        def lane_step(t, st2):
            mb, cnt, thr = st2
            lane = _ffs(mb)
            mb = mb & (mb - 1)
            val = _rot(v, lane)[0]
            ins = cnt & (L - 1)
            cv = cval[_ds16(cnt >> 4)]
            cval[_ds16(cnt >> 4)] = jnp.where(iota == ins, _splat_f(val), cv)
            ci = cidx[_ds16(cnt >> 4)]
            cidx[_ds16(cnt >> 4)] = jnp.where(iota == ins,
                                              _splat_i(base + lane), ci)
            return mb, cnt + 1, thr

        _, cnt, thr = lax.fori_loop(
            0, nsel, lane_step, (packed & jnp.int32(0xFFFF), cnt, thr))
        return cnt, thr

    def do_row(r, _):
        row = wid * ROWS_PER + r
        pltpu.sync_copy(x_hbm.at[row], row_v)

        def g1_step(jq, _):
            accs = []
            for u in range(4):
                j = jq * 4 + u
                acc = row_v[_ds16(j * 8)]
                for t in range(1, 8):
                    acc = jnp.maximum(acc, row_v[_ds16(j * 8 + t)])
                accs.append(acc)
            gv = gmaxv[_ds16(jq >> 2)]
            for u in range(4):
                j = jq * 4 + u
                s1f[_ds16(j)] = accs[u]
                lane = (jq & 3) * 4 + u
                gv = jnp.where(iota == lane, _splat_f(_tmax(accs[u])), gv)
            gmaxv[_ds16(jq >> 2)] = gv
            return 0

        lax.fori_loop(0, NG1 // 4, g1_step, 0)

        def g2_step(q, _):
            acc = s1f[_ds16(q * 8)]
            for t in range(1, 8):
                acc = jnp.maximum(acc, s1f[_ds16(q * 8 + t)])
            s2f[_ds16(q)] = acc
            kbuf[_ds16(q)] = _fkey(acc)
            return 0

        lax.fori_loop(0, NG2, g2_step, 0)

        T0 = bitsearch(kbuf, NG2, jnp.int32(NG2 * L), nbits=16)
        accf = jnp.full((L,), jnp.inf, jnp.float32)
        for q in range(NG2):
            m = kbuf[_ds16(q)] >= T0
            accf = jnp.minimum(accf, jnp.where(m, s2f[_ds16(q)], jnp.inf))
        thr0 = -_tmax(-accf)

        def descend_group(j, st2):
            s1v = s1f[_ds16(j)]
            hp = _packed_mask(jnp.where(s1v >= _splat_f(st2[1]), 1, 0))
            nl = lax.shift_right_arithmetic(hp, 16)

            def one_lane(st3):
                cnt, thr = st3
                lane = _ffs(hp & jnp.int32(0xFFFF))
                perm = (iota + lane) & (L - 1)
                bits = jnp.int32(0)
                nh = jnp.int32(0)
                for t in range(8):
                    val = row_v[_ds16(j * 8 + t)].at[perm].get(
                        mode="promise_in_bounds")[0]
                    hit = jnp.where(val >= thr, 1, 0)
                    bits = bits | lax.shift_left(hit, t)
                    nh = nh + hit

                def hit_step(h, st4):
                    bits, cnt = st4
                    t = _ffs(bits)
                    bits = bits & (bits - 1)
                    val = row_v[_ds16(j * 8 + t)].at[perm].get(
                        mode="promise_in_bounds")[0]
                    ins = cnt & (L - 1)
                    cv = cval[_ds16(cnt >> 4)]
                    cval[_ds16(cnt >> 4)] = jnp.where(
                        iota == ins, _splat_f(val), cv)
                    ci = cidx[_ds16(cnt >> 4)]
                    cidx[_ds16(cnt >> 4)] = jnp.where(
                        iota == ins,
                        _splat_i(j * 128 + t * L + lane), ci)
                    return bits, cnt + 1

                _, cnt = lax.fori_loop(0, nh, hit_step, (bits, cnt))
                return cnt, thr

            def multi_lane(st3):
                for t in range(8):
                    v = row_v[_ds16(j * 8 + t)]
                    st3 = append_hits(v, j * 128 + t * L, st3)
                return st3

            st2 = lax.cond(nl == 1, one_lane, multi_lane, st2)
            return lax.cond(st2[0] >= PRUNE_AT,
                        lambda s5: prune(s5[0]),
                        lambda s5: s5, st2)


        def scan_block(gq, st):
            gv = gmaxv[_ds16(gq)]
            m = jnp.where(gv >= _splat_f(st[1]), 1, 0)
            packed = _packed_mask(m)
            nt = lax.shift_right_arithmetic(packed, 16)

            def grp_step(h, st2):
                bits, cnt, thr = st2
                l = _ffs(bits)
                bits = bits & (bits - 1)
                cnt, thr = descend_group(gq * L + l, (cnt, thr))
                return bits, cnt, thr

            _, cnt, thr = lax.fori_loop(
                0, nt, grp_step,
                (packed & jnp.int32(0xFFFF), st[0], st[1]))
            return cnt, thr

        cnt, thr = lax.fori_loop(0, NG1 // L, scan_block,
                                 (jnp.int32(0), thr0))
        cnt, thr = lax.cond(cnt <= 128,
                            lambda c: prune(c, 8),
                            lambda c: prune(c),
                            cnt)

        for a in range(4):
            rbuf[_ds16(a)] = jnp.zeros((L,), jnp.int32)

        def rank_step(j, _):
            fv = cval[_ds16(j >> 4)]
            fj = _splat_f(_rot(fv, j & (L - 1))[0])
            for a in range(4):
                fa = cval[_ds16(a)]
                pos = iota + a * L
                gt = jnp.where(fj > fa, 1, 0)
                eq = jnp.where(fj == fa, 1, 0)
                lt = jnp.where(pos > j, 1, 0)
                rbuf[_ds16(a)] = rbuf[_ds16(a)] + gt + eq * lt
            return 0

        lax.fori_loop(0, K, rank_step, 0)

        def inv_step(j, _):
            rv = rbuf[_ds16(j >> 4)]
            rj = _rot(rv, j & (L - 1))[0]
            lane_m = jnp.where(iota == _splat_i(rj & (L - 1)), 1, 0)
            for a in range(4):
                ina = jnp.where(lax.shift_right_arithmetic(rj, 4) == a, 1, 0)
                sv = sbuf[_ds16(a)]
                sbuf[_ds16(a)] = jnp.where(lane_m * ina > 0,
                                           _splat_i(j), sv)
            return 0

        lax.fori_loop(0, K, inv_step, 0)

        for a in range(4):
            S = sbuf[_ds16(a)]
            src_lane = S & (L - 1)
            src_vec = lax.shift_right_arithmetic(S, 4)
            ov = jnp.zeros((L,), jnp.float32)
            oi_ = jnp.zeros((L,), jnp.int32)
            for b in range(4):
                g = cval[_ds16(b)].at[src_lane].get(mode="promise_in_bounds")
                gi = cidx[_ds16(b)].at[src_lane].get(mode="promise_in_bounds")
                ov = jnp.where(src_vec == b, g, ov)
                oi_ = jnp.where(src_vec == b, gi, oi_)
            oval[_ds16(a)] = ov
            oidx[_ds16(a)] = oi_

        pltpu.sync_copy(oval, oval_hbm.at[row])
        pltpu.sync_copy(oidx, oidx_hbm.at[row])
        return 0

    lax.fori_loop(0, ROWS_PER, do_row, 0)


_sc_topk = functools.partial(
    pl.kernel,
    out_type=[jax.ShapeDtypeStruct((ROWS, K), jnp.float32),
              jax.ShapeDtypeStruct((ROWS, K), jnp.int32)],
    mesh=plsc.VectorSubcoreMesh(core_axis_name="c", subcore_axis_name="s",
                                num_cores=2, num_subcores=16),
    scratch_types=[
        pltpu.VMEM((N,), jnp.float32),
        pltpu.VMEM((NG1 * L,), jnp.float32),
        pltpu.VMEM((NG2 * L,), jnp.float32),
        pltpu.VMEM((NG2 * L,), jnp.int32),
        pltpu.VMEM((CAP,), jnp.float32),
        pltpu.VMEM((CAP,), jnp.int32),
        pltpu.VMEM((K,), jnp.int32),
        pltpu.VMEM((K,), jnp.int32),
        pltpu.VMEM((K,), jnp.float32),
        pltpu.VMEM((K,), jnp.int32),
        pltpu.VMEM((NG1,), jnp.float32),
    ],
)(_sc_body)


def kernel(input):
    vals, idx = _sc_topk(input)
    return (vals, idx)
